# baseline (speedup 1.0000x reference)
import dataclasses
import functools

import jax
import jax.numpy as jnp
from jax import lax
from jax.experimental import pallas as pl
from jax.experimental.pallas import tpu as pltpu
from jax.experimental.pallas import tpu_sc as plsc

N = 10000
E = 320000
HIDDEN = 128
NG = 50
NL = 6
CUTOFF = 10.0

NC = 2
NS = 16
NW = NC * NS
E_PER_W = E // NW
GW = 80
NBLK = E_PER_W // GW
ROW_STRIDE = 624
ROW_SPAN = 640

@functools.cache
def _mesh():
    return plsc.VectorSubcoreMesh(core_axis_name="c", subcore_axis_name="s",
                                  num_cores=NC, num_subcores=NS)

_SC_CP = pltpu.CompilerParams()
if "needs_layout_passes" in pltpu.CompilerParams.__dataclass_fields__:
    _SC_CP = dataclasses.replace(_SC_CP, needs_layout_passes=False)

_HIGH = jax.lax.Precision.HIGHEST


def _dot(a, b):
    return jnp.dot(a, b, precision=_HIGH, preferred_element_type=jnp.float32)


def _ssp_tc(t):
    return jax.nn.softplus(t) - jnp.log(2.0).astype(jnp.float32)


def _sc_edge_dist(px, py, pz, src, dst):
    @functools.partial(
        pl.kernel,
        out_type=jax.ShapeDtypeStruct((E,), jnp.float32),
        mesh=_mesh(),
        scratch_types=[
            pltpu.VMEM((N,), jnp.float32),
            pltpu.VMEM((N,), jnp.float32),
            pltpu.VMEM((N,), jnp.float32),
            pltpu.VMEM((GW,), jnp.int32),
            pltpu.VMEM((GW,), jnp.int32),
            pltpu.VMEM((GW,), jnp.float32),
        ],
        compiler_params=_SC_CP,
    )
    def k(px_hbm, py_hbm, pz_hbm, src_hbm, dst_hbm, out_hbm,
          px_v, py_v, pz_v, si_v, di_v, o_v):
        wid = lax.axis_index("s") * NC + lax.axis_index("c")
        pltpu.sync_copy(px_hbm, px_v)
        pltpu.sync_copy(py_hbm, py_v)
        pltpu.sync_copy(pz_hbm, pz_v)
        base = wid * E_PER_W

        @pl.loop(0, NBLK)
        def _(i):
            off = base + i * GW
            pltpu.sync_copy(src_hbm.at[pl.ds(off, GW)], si_v)
            pltpu.sync_copy(dst_hbm.at[pl.ds(off, GW)], di_v)

            @pl.loop(0, GW, step=16)
            def _(j):
                sidx = si_v[pl.ds(j, 16)]
                didx = di_v[pl.ds(j, 16)]
                dx = plsc.load_gather(px_v, [sidx]) - plsc.load_gather(px_v, [didx])
                dy = plsc.load_gather(py_v, [sidx]) - plsc.load_gather(py_v, [didx])
                dz = plsc.load_gather(pz_v, [sidx]) - plsc.load_gather(pz_v, [didx])
                o_v[pl.ds(j, 16)] = dx * dx + dy * dy + dz * dz

            pltpu.sync_copy(o_v, out_hbm.at[pl.ds(off, GW)])

    return k(px, py, pz, src, dst)


def _sc_layer(xl, wf_all, layer, src, dst, zeros_nd):
    @functools.partial(
        pl.kernel,
        out_type=jax.ShapeDtypeStruct((NC, N, HIDDEN), jnp.float32),
        mesh=_mesh(),
        scratch_types=[
            pltpu.VMEM((GW,), jnp.int32),
            pltpu.VMEM((GW,), jnp.int32),
            pltpu.VMEM((GW,), jnp.int32),
            pltpu.VMEM((GW,), jnp.int32),
            pltpu.VMEM((GW, HIDDEN), jnp.float32),
            pltpu.VMEM((GW, HIDDEN), jnp.float32),
            pltpu.VMEM((GW, HIDDEN), jnp.float32),
            pltpu.VMEM((GW, HIDDEN), jnp.float32),
            pltpu.VMEM_SHARED((N, HIDDEN), jnp.float32),
            pltpu.SemaphoreType.DMA,
            pltpu.SemaphoreType.DMA,
        ],
    )
    def k(xl_hbm, wf_hbm, src_hbm, dst_hbm, zero_hbm, out_hbm,
          si0, si1, di0, di1, wf0, wf1, gx0, gx1, acc_sh, sem0, sem1):
        si = (si0, si1)
        di = (di0, di1)
        wfv = (wf0, wf1)
        gxv = (gx0, gx1)
        sem = (sem0, sem1)
        cid = lax.axis_index("c")
        sid = lax.axis_index("s")
        wid = sid * NC + cid
        r0 = sid * ROW_STRIDE
        pltpu.sync_copy(zero_hbm.at[pl.ds(r0, ROW_SPAN)],
                        acc_sh.at[pl.ds(r0, ROW_SPAN)])
        plsc.subcore_barrier()
        base = wid * E_PER_W

        def issue(blk, p):
            off = base + blk * GW
            pltpu.sync_copy(src_hbm.at[pl.ds(off, GW)], si[p])
            pltpu.async_copy(wf_hbm.at[layer, pl.ds(off, GW)], wfv[p], sem[p])
            pltpu.async_copy(xl_hbm.at[si[p]], gxv[p], sem[p])

        issue(0, 0)
        issue(1, 1)

        @pl.loop(0, NBLK + 1, step=2)
        def _(i):
            for p in range(2):
                blk = i + p

                @pl.when(blk < NBLK)
                def _():
                    pltpu.make_async_copy(
                        wf_hbm.at[layer, pl.ds(base, GW)], wfv[p], sem[p]).wait()
                    pltpu.make_async_copy(
                        xl_hbm.at[si[p]], gxv[p], sem[p]).wait()
                    pltpu.sync_copy(dst_hbm.at[pl.ds(base + blk * GW, GW)],
                                    di[p])

                    @plsc.parallel_loop(0, GW, unroll=4)
                    def _(r):
                        @pl.loop(0, HIDDEN, step=16, unroll=8)
                        def _(c):
                            gxv[p][r, pl.ds(c, 16)] = (
                                gxv[p][r, pl.ds(c, 16)]
                                * wfv[p][r, pl.ds(c, 16)])

                    pltpu.sync_copy(gxv[p], acc_sh.at[di[p]], add=True)

                    @pl.when(blk + 2 < NBLK)
                    def _():
                        issue(blk + 2, p)

        plsc.subcore_barrier()
        pltpu.sync_copy(acc_sh.at[pl.ds(r0, ROW_SPAN)],
                        out_hbm.at[cid, pl.ds(r0, ROW_SPAN)])

    return k(xl, wf_all, src, dst, zeros_nd)


_BE = 2000
_BN = 2000


def _tc_wf_all(d2, Wm1, bm1, Wm2, bm2):
    step = CUTOFF / (NG - 1)
    coeff = -0.5 / step**2

    def body(d2_ref, w1_ref, b1_ref, w2_ref, b2_ref, out_ref):
        ew = jnp.sqrt(d2_ref[...] + 1e-12)
        cc = 0.5 * (jnp.cos(ew * (jnp.pi / CUTOFF)) + 1.0)
        gau = lax.broadcasted_iota(jnp.int32, (_BE, NG), 1).astype(
            jnp.float32) * step
        ea = jnp.exp(coeff * (ew - gau) ** 2)
        t = _ssp_tc(_dot(ea, w1_ref[0]) + b1_ref[0])
        out_ref[0] = (_dot(t, w2_ref[0]) + b2_ref[0]) * cc

    return pl.pallas_call(
        body,
        grid=(NL, E // _BE),
        in_specs=[
            pl.BlockSpec((_BE, 1), lambda l, i: (i, 0)),
            pl.BlockSpec((1, NG, HIDDEN), lambda l, i: (l, 0, 0)),
            pl.BlockSpec((1, 1, HIDDEN), lambda l, i: (l, 0, 0)),
            pl.BlockSpec((1, HIDDEN, HIDDEN), lambda l, i: (l, 0, 0)),
            pl.BlockSpec((1, 1, HIDDEN), lambda l, i: (l, 0, 0)),
        ],
        out_specs=pl.BlockSpec((1, _BE, HIDDEN), lambda l, i: (l, i, 0)),
        out_shape=jax.ShapeDtypeStruct((NL, E, HIDDEN), jnp.float32),
    )(d2, Wm1, bm1.reshape(NL, 1, HIDDEN), Wm2, bm2.reshape(NL, 1, HIDDEN))


def _tc_embed(x, W_emb, b_emb, Wc1_0):
    def body(x_ref, we_ref, be_ref, wc_ref, h_ref, xl_ref):
        h = _dot(x_ref[...], we_ref[...]) + be_ref[...]
        h_ref[...] = h
        xl_ref[...] = _dot(h, wc_ref[...])

    return pl.pallas_call(
        body,
        grid=(N // _BN,),
        in_specs=[
            pl.BlockSpec((_BN, HIDDEN), lambda i: (i, 0)),
            pl.BlockSpec((HIDDEN, HIDDEN), lambda i: (0, 0)),
            pl.BlockSpec((1, HIDDEN), lambda i: (0, 0)),
            pl.BlockSpec((HIDDEN, HIDDEN), lambda i: (0, 0)),
        ],
        out_specs=[
            pl.BlockSpec((_BN, HIDDEN), lambda i: (i, 0)),
            pl.BlockSpec((_BN, HIDDEN), lambda i: (i, 0)),
        ],
        out_shape=[
            jax.ShapeDtypeStruct((N, HIDDEN), jnp.float32),
            jax.ShapeDtypeStruct((N, HIDDEN), jnp.float32),
        ],
    )(x, W_emb, b_emb.reshape(1, HIDDEN), Wc1_0)


def _tc_node(parts, h, Wc2_l, bc2_l, Wi_l, bi_l, Wc1_next):
    def body(p_ref, h_ref, wc2_ref, bc2_ref, wi_ref, bi_ref, wn_ref,
             h_out, xl_out):
        agg = p_ref[0] + p_ref[1]
        xc = _dot(_ssp_tc(_dot(agg, wc2_ref[...]) + bc2_ref[...]),
                  wi_ref[...]) + bi_ref[...]
        hn = h_ref[...] + xc
        h_out[...] = hn
        xl_out[...] = _dot(hn, wn_ref[...])

    return pl.pallas_call(
        body,
        grid=(N // _BN,),
        in_specs=[
            pl.BlockSpec((2, _BN, HIDDEN), lambda i: (0, i, 0)),
            pl.BlockSpec((_BN, HIDDEN), lambda i: (i, 0)),
            pl.BlockSpec((HIDDEN, HIDDEN), lambda i: (0, 0)),
            pl.BlockSpec((1, HIDDEN), lambda i: (0, 0)),
            pl.BlockSpec((HIDDEN, HIDDEN), lambda i: (0, 0)),
            pl.BlockSpec((1, HIDDEN), lambda i: (0, 0)),
            pl.BlockSpec((HIDDEN, HIDDEN), lambda i: (0, 0)),
        ],
        out_specs=[
            pl.BlockSpec((_BN, HIDDEN), lambda i: (i, 0)),
            pl.BlockSpec((_BN, HIDDEN), lambda i: (i, 0)),
        ],
        out_shape=[
            jax.ShapeDtypeStruct((N, HIDDEN), jnp.float32),
            jax.ShapeDtypeStruct((N, HIDDEN), jnp.float32),
        ],
    )(parts, h, Wc2_l, bc2_l.reshape(1, HIDDEN), Wi_l, bi_l.reshape(1, HIDDEN),
      Wc1_next)


def _tc_node_final(parts, h, Wc2_l, bc2_l, Wi_l, bi_l, W1, b1, W2, b2, Wd, bd):
    def body(p_ref, h_ref, wc2_ref, bc2_ref, wi_ref, bi_ref,
             w1_ref, b1_ref, w2_ref, b2_ref, wd_ref, bd_ref, out_ref):
        agg = p_ref[0] + p_ref[1]
        xc = _dot(_ssp_tc(_dot(agg, wc2_ref[...]) + bc2_ref[...]),
                  wi_ref[...]) + bi_ref[...]
        hn = h_ref[...] + xc
        ne = _dot(_ssp_tc(_dot(hn, w1_ref[...]) + b1_ref[...]),
                  w2_ref[...]) + b2_ref[...]
        out_ref[...] = _dot(ne, wd_ref[...]) + bd_ref[...]

    ncls = Wd.shape[1]
    hh = W1.shape[1]
    out = W2.shape[1]
    return pl.pallas_call(
        body,
        grid=(N // _BN,),
        in_specs=[
            pl.BlockSpec((2, _BN, HIDDEN), lambda i: (0, i, 0)),
            pl.BlockSpec((_BN, HIDDEN), lambda i: (i, 0)),
            pl.BlockSpec((HIDDEN, HIDDEN), lambda i: (0, 0)),
            pl.BlockSpec((1, HIDDEN), lambda i: (0, 0)),
            pl.BlockSpec((HIDDEN, HIDDEN), lambda i: (0, 0)),
            pl.BlockSpec((1, HIDDEN), lambda i: (0, 0)),
            pl.BlockSpec((HIDDEN, hh), lambda i: (0, 0)),
            pl.BlockSpec((1, hh), lambda i: (0, 0)),
            pl.BlockSpec((hh, out), lambda i: (0, 0)),
            pl.BlockSpec((1, out), lambda i: (0, 0)),
            pl.BlockSpec((out, ncls), lambda i: (0, 0)),
            pl.BlockSpec((1, ncls), lambda i: (0, 0)),
        ],
        out_specs=pl.BlockSpec((_BN, ncls), lambda i: (i, 0)),
        out_shape=jax.ShapeDtypeStruct((N, ncls), jnp.float32),
    )(parts, h, Wc2_l, bc2_l.reshape(1, HIDDEN), Wi_l, bi_l.reshape(1, HIDDEN),
      W1, b1.reshape(1, -1), W2, b2.reshape(1, -1), Wd, bd.reshape(1, -1))


def kernel(x, pos, edge_index, batch, W_emb, b_emb, Wm1, bm1, Wm2, bm2,
           Wc1, Wc2, bc2, Wi, bi, W1, b1, W2, b2, Wd, bd):
    del batch
    src = edge_index[0]
    dst = edge_index[1]

    px, py, pz = pos[:, 0], pos[:, 1], pos[:, 2]
    d2 = _sc_edge_dist(px, py, pz, src, dst)
    wf_all = _tc_wf_all(d2.reshape(E, 1), Wm1, bm1, Wm2, bm2)

    h, xl = _tc_embed(x, W_emb, b_emb, Wc1[0])

    zeros_nd = jnp.zeros((N, HIDDEN), jnp.float32)
    for l in range(NL):
        parts = _sc_layer(xl, wf_all, l, src, dst, zeros_nd)
        if l < NL - 1:
            h, xl = _tc_node(parts, h, Wc2[l], bc2[l], Wi[l], bi[l], Wc1[l + 1])
        else:
            pred = _tc_node_final(parts, h, Wc2[l], bc2[l], Wi[l], bi[l],
                                  W1, b1, W2, b2, Wd, bd)
    return pred


# speedup vs baseline: 3.0647x; 3.0647x over previous
import dataclasses
import functools

import jax
import jax.numpy as jnp
from jax import lax
from jax.experimental import pallas as pl
from jax.experimental.pallas import tpu as pltpu
from jax.experimental.pallas import tpu_sc as plsc

N = 10000
E = 320000
HIDDEN = 128
NG = 50
NL = 6
CUTOFF = 10.0

NC = 2
NS = 16
NW = NC * NS
E_PER_W = E // NW
GW = 80
NBLK = E_PER_W // GW
ROW_STRIDE = 624
ROW_SPAN = 640

@functools.cache
def _mesh():
    return plsc.VectorSubcoreMesh(core_axis_name="c", subcore_axis_name="s",
                                  num_cores=NC, num_subcores=NS)

_SC_CP = pltpu.CompilerParams()
if "needs_layout_passes" in pltpu.CompilerParams.__dataclass_fields__:
    _SC_CP = dataclasses.replace(_SC_CP, needs_layout_passes=False)

_HIGH = jax.lax.Precision.DEFAULT


def _dot(a, b):
    return jnp.dot(a, b, precision=_HIGH, preferred_element_type=jnp.float32)


def _ssp_tc(t):
    return jax.nn.softplus(t) - jnp.log(2.0).astype(jnp.float32)


def _sc_edge_dist(px, py, pz, src, dst):
    @functools.partial(
        pl.kernel,
        out_type=jax.ShapeDtypeStruct((E,), jnp.float32),
        mesh=_mesh(),
        scratch_types=[
            pltpu.VMEM((N,), jnp.float32),
            pltpu.VMEM((N,), jnp.float32),
            pltpu.VMEM((N,), jnp.float32),
            pltpu.VMEM((GW,), jnp.int32),
            pltpu.VMEM((GW,), jnp.int32),
            pltpu.VMEM((GW,), jnp.float32),
        ],
        compiler_params=_SC_CP,
    )
    def k(px_hbm, py_hbm, pz_hbm, src_hbm, dst_hbm, out_hbm,
          px_v, py_v, pz_v, si_v, di_v, o_v):
        wid = lax.axis_index("s") * NC + lax.axis_index("c")
        pltpu.sync_copy(px_hbm, px_v)
        pltpu.sync_copy(py_hbm, py_v)
        pltpu.sync_copy(pz_hbm, pz_v)
        base = wid * E_PER_W

        @pl.loop(0, NBLK)
        def _(i):
            off = base + i * GW
            pltpu.sync_copy(src_hbm.at[pl.ds(off, GW)], si_v)
            pltpu.sync_copy(dst_hbm.at[pl.ds(off, GW)], di_v)

            @pl.loop(0, GW, step=16)
            def _(j):
                sidx = si_v[pl.ds(j, 16)]
                didx = di_v[pl.ds(j, 16)]
                dx = plsc.load_gather(px_v, [sidx]) - plsc.load_gather(px_v, [didx])
                dy = plsc.load_gather(py_v, [sidx]) - plsc.load_gather(py_v, [didx])
                dz = plsc.load_gather(pz_v, [sidx]) - plsc.load_gather(pz_v, [didx])
                o_v[pl.ds(j, 16)] = dx * dx + dy * dy + dz * dz

            pltpu.sync_copy(o_v, out_hbm.at[pl.ds(off, GW)])

    return k(px, py, pz, src, dst)


def _sc_layer(xl, wf, src, dst, zeros_nd):
    @functools.partial(
        pl.kernel,
        out_type=jax.ShapeDtypeStruct((NC, N, HIDDEN), jnp.float32),
        mesh=_mesh(),
        scratch_types=[
            pltpu.VMEM((GW,), jnp.int32),
            pltpu.VMEM((GW,), jnp.int32),
            pltpu.VMEM((GW,), jnp.int32),
            pltpu.VMEM((GW,), jnp.int32),
            pltpu.VMEM((GW, HIDDEN), jnp.float32),
            pltpu.VMEM((GW, HIDDEN), jnp.float32),
            pltpu.VMEM((GW, HIDDEN), jnp.float32),
            pltpu.VMEM((GW, HIDDEN), jnp.float32),
            pltpu.VMEM_SHARED((N, HIDDEN), jnp.float32),
            pltpu.SemaphoreType.DMA,
            pltpu.SemaphoreType.DMA,
        ],
    )
    def k(xl_hbm, wf_hbm, src_hbm, dst_hbm, zero_hbm, out_hbm,
          si0, si1, di0, di1, wf0, wf1, gx0, gx1, acc_sh, sem0, sem1):
        si = (si0, si1)
        di = (di0, di1)
        wfv = (wf0, wf1)
        gxv = (gx0, gx1)
        sem = (sem0, sem1)
        cid = lax.axis_index("c")
        sid = lax.axis_index("s")
        wid = sid * NC + cid
        r0 = sid * ROW_STRIDE
        pltpu.sync_copy(zero_hbm.at[pl.ds(r0, ROW_SPAN)],
                        acc_sh.at[pl.ds(r0, ROW_SPAN)])
        plsc.subcore_barrier()
        base = wid * E_PER_W

        def issue(blk, p):
            off = base + blk * GW
            pltpu.sync_copy(src_hbm.at[pl.ds(off, GW)], si[p])
            pltpu.async_copy(wf_hbm.at[pl.ds(off, GW)], wfv[p], sem[p])
            pltpu.async_copy(xl_hbm.at[si[p]], gxv[p], sem[p])

        issue(0, 0)
        issue(1, 1)

        @pl.loop(0, NBLK + 1, step=2)
        def _(i):
            for p in range(2):
                blk = i + p

                @pl.when(blk < NBLK)
                def _():
                    pltpu.make_async_copy(
                        wf_hbm.at[pl.ds(base, GW)], wfv[p], sem[p]).wait()
                    pltpu.make_async_copy(
                        xl_hbm.at[si[p]], gxv[p], sem[p]).wait()
                    pltpu.sync_copy(dst_hbm.at[pl.ds(base + blk * GW, GW)],
                                    di[p])

                    @pl.loop(0, GW)
                    def _(r):
                        @pl.loop(0, HIDDEN, step=16, unroll=8)
                        def _(c):
                            gxv[p][r, pl.ds(c, 16)] = (
                                gxv[p][r, pl.ds(c, 16)]
                                * wfv[p][r, pl.ds(c, 16)])

                    pltpu.sync_copy(gxv[p], acc_sh.at[di[p]], add=True)

                    @pl.when(blk + 2 < NBLK)
                    def _():
                        issue(blk + 2, p)

        plsc.subcore_barrier()
        pltpu.sync_copy(acc_sh.at[pl.ds(r0, ROW_SPAN)],
                        out_hbm.at[cid, pl.ds(r0, ROW_SPAN)])

    return k(xl, wf, src, dst, zeros_nd)


_BE = 2000
_BN = 2000


def _tc_edge_prep(d2):
    def body(d2_ref, es_ref):
        ew = jnp.sqrt(d2_ref[...] + 1e-12)
        cc = 0.5 * (jnp.cos(ew * (jnp.pi / CUTOFF)) + 1.0)
        es_ref[...] = jnp.concatenate([ew, cc], axis=-1)

    return pl.pallas_call(
        body,
        grid=(E // _BE,),
        in_specs=[pl.BlockSpec((_BE, 1), lambda i: (i, 0))],
        out_specs=pl.BlockSpec((_BE, 2), lambda i: (i, 0)),
        out_shape=jax.ShapeDtypeStruct((E, 2), jnp.float32),
    )(d2)


def _tc_embed(x, W_emb, b_emb, Wc1_0):
    def body(x_ref, we_ref, be_ref, wc_ref, h_ref, xl_ref):
        h = _dot(x_ref[...], we_ref[...]) + be_ref[...]
        h_ref[...] = h
        xl_ref[...] = _dot(h, wc_ref[...])

    return pl.pallas_call(
        body,
        grid=(N // _BN,),
        in_specs=[
            pl.BlockSpec((_BN, HIDDEN), lambda i: (i, 0)),
            pl.BlockSpec((HIDDEN, HIDDEN), lambda i: (0, 0)),
            pl.BlockSpec((1, HIDDEN), lambda i: (0, 0)),
            pl.BlockSpec((HIDDEN, HIDDEN), lambda i: (0, 0)),
        ],
        out_specs=[
            pl.BlockSpec((_BN, HIDDEN), lambda i: (i, 0)),
            pl.BlockSpec((_BN, HIDDEN), lambda i: (i, 0)),
        ],
        out_shape=[
            jax.ShapeDtypeStruct((N, HIDDEN), jnp.float32),
            jax.ShapeDtypeStruct((N, HIDDEN), jnp.float32),
        ],
    )(x, W_emb, b_emb.reshape(1, HIDDEN), Wc1_0)


def _tc_edge_wf(es, Wm1_l, bm1_l, Wm2_l, bm2_l):
    step = CUTOFF / (NG - 1)
    coeff = -0.5 / step**2

    def body(es_ref, w1_ref, b1_ref, w2_ref, b2_ref, out_ref):
        ew = es_ref[:, 0:1]
        cc = es_ref[:, 1:2]
        gau = lax.broadcasted_iota(jnp.int32, (_BE, NG), 1).astype(
            jnp.float32) * step
        ea = jnp.exp(coeff * (ew - gau) ** 2)
        t = _ssp_tc(_dot(ea, w1_ref[...]) + b1_ref[...])
        out_ref[...] = (_dot(t, w2_ref[...]) + b2_ref[...]) * cc

    return pl.pallas_call(
        body,
        grid=(E // _BE,),
        in_specs=[
            pl.BlockSpec((_BE, 2), lambda i: (i, 0)),
            pl.BlockSpec((NG, HIDDEN), lambda i: (0, 0)),
            pl.BlockSpec((1, HIDDEN), lambda i: (0, 0)),
            pl.BlockSpec((HIDDEN, HIDDEN), lambda i: (0, 0)),
            pl.BlockSpec((1, HIDDEN), lambda i: (0, 0)),
        ],
        out_specs=pl.BlockSpec((_BE, HIDDEN), lambda i: (i, 0)),
        out_shape=jax.ShapeDtypeStruct((E, HIDDEN), jnp.float32),
    )(es, Wm1_l, bm1_l.reshape(1, HIDDEN), Wm2_l, bm2_l.reshape(1, HIDDEN))


def _tc_node(parts, h, Wc2_l, bc2_l, Wi_l, bi_l, Wc1_next):
    def body(p_ref, h_ref, wc2_ref, bc2_ref, wi_ref, bi_ref, wn_ref,
             h_out, xl_out):
        agg = p_ref[0] + p_ref[1]
        xc = _dot(_ssp_tc(_dot(agg, wc2_ref[...]) + bc2_ref[...]),
                  wi_ref[...]) + bi_ref[...]
        hn = h_ref[...] + xc
        h_out[...] = hn
        xl_out[...] = _dot(hn, wn_ref[...])

    return pl.pallas_call(
        body,
        grid=(N // _BN,),
        in_specs=[
            pl.BlockSpec((2, _BN, HIDDEN), lambda i: (0, i, 0)),
            pl.BlockSpec((_BN, HIDDEN), lambda i: (i, 0)),
            pl.BlockSpec((HIDDEN, HIDDEN), lambda i: (0, 0)),
            pl.BlockSpec((1, HIDDEN), lambda i: (0, 0)),
            pl.BlockSpec((HIDDEN, HIDDEN), lambda i: (0, 0)),
            pl.BlockSpec((1, HIDDEN), lambda i: (0, 0)),
            pl.BlockSpec((HIDDEN, HIDDEN), lambda i: (0, 0)),
        ],
        out_specs=[
            pl.BlockSpec((_BN, HIDDEN), lambda i: (i, 0)),
            pl.BlockSpec((_BN, HIDDEN), lambda i: (i, 0)),
        ],
        out_shape=[
            jax.ShapeDtypeStruct((N, HIDDEN), jnp.float32),
            jax.ShapeDtypeStruct((N, HIDDEN), jnp.float32),
        ],
    )(parts, h, Wc2_l, bc2_l.reshape(1, HIDDEN), Wi_l, bi_l.reshape(1, HIDDEN),
      Wc1_next)


def _tc_node_final(parts, h, Wc2_l, bc2_l, Wi_l, bi_l, W1, b1, W2, b2, Wd, bd):
    def body(p_ref, h_ref, wc2_ref, bc2_ref, wi_ref, bi_ref,
             w1_ref, b1_ref, w2_ref, b2_ref, wd_ref, bd_ref, out_ref):
        agg = p_ref[0] + p_ref[1]
        xc = _dot(_ssp_tc(_dot(agg, wc2_ref[...]) + bc2_ref[...]),
                  wi_ref[...]) + bi_ref[...]
        hn = h_ref[...] + xc
        ne = _dot(_ssp_tc(_dot(hn, w1_ref[...]) + b1_ref[...]),
                  w2_ref[...]) + b2_ref[...]
        out_ref[...] = _dot(ne, wd_ref[...]) + bd_ref[...]

    ncls = Wd.shape[1]
    hh = W1.shape[1]
    out = W2.shape[1]
    return pl.pallas_call(
        body,
        grid=(N // _BN,),
        in_specs=[
            pl.BlockSpec((2, _BN, HIDDEN), lambda i: (0, i, 0)),
            pl.BlockSpec((_BN, HIDDEN), lambda i: (i, 0)),
            pl.BlockSpec((HIDDEN, HIDDEN), lambda i: (0, 0)),
            pl.BlockSpec((1, HIDDEN), lambda i: (0, 0)),
            pl.BlockSpec((HIDDEN, HIDDEN), lambda i: (0, 0)),
            pl.BlockSpec((1, HIDDEN), lambda i: (0, 0)),
            pl.BlockSpec((HIDDEN, hh), lambda i: (0, 0)),
            pl.BlockSpec((1, hh), lambda i: (0, 0)),
            pl.BlockSpec((hh, out), lambda i: (0, 0)),
            pl.BlockSpec((1, out), lambda i: (0, 0)),
            pl.BlockSpec((out, ncls), lambda i: (0, 0)),
            pl.BlockSpec((1, ncls), lambda i: (0, 0)),
        ],
        out_specs=pl.BlockSpec((_BN, ncls), lambda i: (i, 0)),
        out_shape=jax.ShapeDtypeStruct((N, ncls), jnp.float32),
    )(parts, h, Wc2_l, bc2_l.reshape(1, HIDDEN), Wi_l, bi_l.reshape(1, HIDDEN),
      W1, b1.reshape(1, -1), W2, b2.reshape(1, -1), Wd, bd.reshape(1, -1))


def kernel(x, pos, edge_index, batch, W_emb, b_emb, Wm1, bm1, Wm2, bm2,
           Wc1, Wc2, bc2, Wi, bi, W1, b1, W2, b2, Wd, bd):
    del batch
    src = edge_index[0]
    dst = edge_index[1]

    px, py, pz = pos[:, 0], pos[:, 1], pos[:, 2]
    d2 = _sc_edge_dist(px, py, pz, src, dst)
    es = _tc_edge_prep(d2.reshape(E, 1))

    h, xl = _tc_embed(x, W_emb, b_emb, Wc1[0])

    zeros_nd = jnp.zeros((N, HIDDEN), jnp.float32)
    for l in range(NL):
        wf = _tc_edge_wf(es, Wm1[l], bm1[l], Wm2[l], bm2[l])
        parts = _sc_layer(xl, wf, src, dst, zeros_nd)
        if l < NL - 1:
            h, xl = _tc_node(parts, h, Wc2[l], bc2[l], Wi[l], bi[l], Wc1[l + 1])
        else:
            pred = _tc_node_final(parts, h, Wc2[l], bc2[l], Wi[l], bi[l],
                                  W1, b1, W2, b2, Wd, bd)
    return pred
